# baseline (speedup 1.0000x reference)
import jax
import jax.numpy as jnp
from jax import lax
from jax.experimental import pallas as pl
from jax.experimental.pallas import tpu as pltpu

ROWS, COLS = 128, 32768
CR = 8
NCHUNK = ROWS // CR
NBUF = 6


def _tc_body(x_hbm, o_ref, buf, *sems):

    def start(c):
        return pltpu.make_async_copy(
            x_hbm.at[pl.ds(c * CR, CR), :], buf.at[c % NBUF], sems[c % NBUF]
        )

    for p in range(NBUF - 1):
        start(p).start()
    iota = lax.broadcasted_iota(jnp.int32, (CR, COLS), 1)

    for c in range(NCHUNK):
        if c + NBUF - 1 < NCHUNK:
            start(c + NBUF - 1).start()
        start(c).wait()
        xb = buf[c % NBUF]
        m = jnp.max(xb, axis=1, keepdims=True)
        idx = jnp.where(xb == m, iota, COLS)
        o_ref[pl.ds(c * CR, CR)] = jnp.min(idx, axis=1)


def _argmax_tc(x):
    return pl.pallas_call(
        _tc_body,
        in_specs=[pl.BlockSpec(memory_space=pl.ANY)],
        out_specs=pl.BlockSpec(memory_space=pltpu.MemorySpace.VMEM),
        out_shape=jax.ShapeDtypeStruct((ROWS,), jnp.int32),
        scratch_shapes=[
            pltpu.VMEM((NBUF, CR, COLS), jnp.float32),
        ] + [pltpu.SemaphoreType.DMA] * NBUF + [
        ],
    )(x)


def kernel(x):
    return _argmax_tc(x)


# speedup vs baseline: 1.7377x; 1.7377x over previous
import jax
import jax.numpy as jnp
from jax import lax
from jax.experimental import pallas as pl
from jax.experimental.pallas import tpu as pltpu

ROWS, COLS = 128, 32768
CR = 32
NCHUNK = ROWS // CR
NBUF = 4


def _tc_body(x_hbm, o_ref, buf, *sems):

    def start(c):
        return pltpu.make_async_copy(
            x_hbm.at[pl.ds(c * CR, CR), :], buf.at[c % NBUF], sems[c % NBUF]
        )

    for p in range(NBUF - 1):
        start(p).start()
    iota = lax.broadcasted_iota(jnp.int32, (CR, COLS), 1)

    for c in range(NCHUNK):
        if c + NBUF - 1 < NCHUNK:
            start(c + NBUF - 1).start()
        start(c).wait()
        xb = buf[c % NBUF]
        m = jnp.max(xb, axis=1, keepdims=True)
        idx = jnp.where(xb == m, iota, COLS)
        o_ref[pl.ds(c * CR, CR)] = jnp.min(idx, axis=1)


def _argmax_tc(x):
    return pl.pallas_call(
        _tc_body,
        in_specs=[pl.BlockSpec(memory_space=pl.ANY)],
        out_specs=pl.BlockSpec(memory_space=pltpu.MemorySpace.VMEM),
        out_shape=jax.ShapeDtypeStruct((ROWS,), jnp.int32),
        scratch_shapes=[
            pltpu.VMEM((NBUF, CR, COLS), jnp.float32),
        ] + [pltpu.SemaphoreType.DMA] * NBUF + [
        ],
    )(x)


def kernel(x):
    return _argmax_tc(x)
